# Initial kernel scaffold; baseline (speedup 1.0000x reference)
import jax
import jax.numpy as jnp
from jax.experimental import pallas as pl


def kernel(x, edge_index, batch, emb1, emb2, w1_l0, b1_l0, w2_l0, b2_l0, w1_l1, b1_l1, w2_l1, b2_l1, w1_l2, b1_l2, w2_l2, b2_l2):
    raise NotImplementedError("write your pallas kernel here")


# speedup vs baseline: 5.1379x; 5.1379x over previous
import functools

import jax
import jax.numpy as jnp
from jax import lax
from jax.experimental import pallas as pl
from jax.experimental.pallas import tpu as pltpu, tpu_sc as plsc

N = 10000
E = 160000
D = 256
DH = 128
H = 512
G = 64
A1 = 120
A2 = 3

NB = 1000
GRID = N // NB

TILES = 16
EPT = E // TILES
CHUNK = 80
CHUNKS = EPT // CHUNK
RPT = 624
RPT_LAST = N - (TILES - 1) * RPT


def _sc_agg_body(y0_hbm, y1_hbm, src_hbm, dst_hbm, zero_hbm, out0_hbm, out1_hbm,
                 src_v, dst_v, rows_v, agg_sh, sem):
    c = lax.axis_index("c")
    s = lax.axis_index("s")
    base = pl.multiple_of(s * RPT, 8)

    @pl.when(s < TILES - 1)
    def _():
        pltpu.sync_copy(zero_hbm.at[pl.ds(base, RPT)],
                        agg_sh.at[pl.ds(base, RPT)])

    @pl.when(s == TILES - 1)
    def _():
        pltpu.sync_copy(zero_hbm.at[pl.ds((TILES - 1) * RPT, RPT_LAST)],
                        agg_sh.at[pl.ds((TILES - 1) * RPT, RPT_LAST)])

    pltpu.sync_copy(src_hbm.at[s], src_v)
    pltpu.sync_copy(dst_hbm.at[s], dst_v)
    plsc.subcore_barrier()

    def run(table_hbm):
        def step(j, carry):
            pltpu.async_copy(table_hbm.at[src_v.at[j]], rows_v, sem).wait()
            pltpu.sync_copy(rows_v, agg_sh.at[dst_v.at[j]], add=True)
            return carry
        lax.fori_loop(0, CHUNKS, step, 0, unroll=False)

    @pl.when(c == 0)
    def _():
        run(y0_hbm)

    @pl.when(c == 1)
    def _():
        run(y1_hbm)

    plsc.subcore_barrier()

    def writeback(out_hbm):
        @pl.when(s < TILES - 1)
        def _():
            pltpu.sync_copy(agg_sh.at[pl.ds(base, RPT)],
                            out_hbm.at[pl.ds(base, RPT)])

        @pl.when(s == TILES - 1)
        def _():
            pltpu.sync_copy(agg_sh.at[pl.ds((TILES - 1) * RPT, RPT_LAST)],
                            out_hbm.at[pl.ds((TILES - 1) * RPT, RPT_LAST)])

    @pl.when(c == 0)
    def _():
        writeback(out0_hbm)

    @pl.when(c == 1)
    def _():
        writeback(out1_hbm)


@functools.cache
def _get_sc_agg():
    mesh = plsc.VectorSubcoreMesh(core_axis_name="c", subcore_axis_name="s")
    return pl.kernel(
        _sc_agg_body,
        mesh=mesh,
        out_type=(
            jax.ShapeDtypeStruct((N, DH), jnp.float32),
            jax.ShapeDtypeStruct((N, DH), jnp.float32),
        ),
        scratch_types=[
            pltpu.VMEM((CHUNKS, CHUNK), jnp.int32),
            pltpu.VMEM((CHUNKS, CHUNK), jnp.int32),
            pltpu.VMEM((CHUNK, DH), jnp.float32),
            pltpu.VMEM_SHARED((N, DH), jnp.float32),
            pltpu.SemaphoreType.DMA,
        ],
    )


def _embed_body(x_ref, e1_ref, e2_ref, y0_ref, y1_ref):
    xb = x_ref[...]
    i1 = xb[:, 0:1]
    i2 = xb[:, 1:2]
    oh1 = (i1 == lax.broadcasted_iota(jnp.int32, (NB, A1), 1)).astype(jnp.float32)
    oh2 = (i2 == lax.broadcasted_iota(jnp.int32, (NB, A2), 1)).astype(jnp.float32)
    y = oh1 @ e1_ref[...] + oh2 @ e2_ref[...]
    y0_ref[...] = y[:, :DH]
    y1_ref[...] = y[:, DH:]


_embed = pl.pallas_call(
    _embed_body,
    grid=(GRID,),
    in_specs=[
        pl.BlockSpec((NB, 2), lambda i: (i, 0)),
        pl.BlockSpec((A1, D), lambda i: (0, 0)),
        pl.BlockSpec((A2, D), lambda i: (0, 0)),
    ],
    out_specs=[
        pl.BlockSpec((NB, DH), lambda i: (i, 0)),
        pl.BlockSpec((NB, DH), lambda i: (i, 0)),
    ],
    out_shape=[
        jax.ShapeDtypeStruct((N, DH), jnp.float32),
        jax.ShapeDtypeStruct((N, DH), jnp.float32),
    ],
)


def _mlp_body(y0_ref, y1_ref, a0_ref, a1_ref, w1_ref, b1_ref, w2_ref, b2_ref,
              o0_ref, o1_ref):
    h = jnp.concatenate(
        [y0_ref[...] + a0_ref[...], y1_ref[...] + a1_ref[...]], axis=1)
    t = jnp.maximum(h @ w1_ref[...] + b1_ref[...], 0.0)
    o = jnp.maximum(t @ w2_ref[...] + b2_ref[...], 0.0)
    o0_ref[...] = o[:, :DH]
    o1_ref[...] = o[:, DH:]


_mlp = pl.pallas_call(
    _mlp_body,
    grid=(GRID,),
    in_specs=[
        pl.BlockSpec((NB, DH), lambda i: (i, 0)),
        pl.BlockSpec((NB, DH), lambda i: (i, 0)),
        pl.BlockSpec((NB, DH), lambda i: (i, 0)),
        pl.BlockSpec((NB, DH), lambda i: (i, 0)),
        pl.BlockSpec((D, H), lambda i: (0, 0)),
        pl.BlockSpec((1, H), lambda i: (0, 0)),
        pl.BlockSpec((H, D), lambda i: (0, 0)),
        pl.BlockSpec((1, D), lambda i: (0, 0)),
    ],
    out_specs=[
        pl.BlockSpec((NB, DH), lambda i: (i, 0)),
        pl.BlockSpec((NB, DH), lambda i: (i, 0)),
    ],
    out_shape=[
        jax.ShapeDtypeStruct((N, DH), jnp.float32),
        jax.ShapeDtypeStruct((N, DH), jnp.float32),
    ],
)


def _mlp_pool_body(y0_ref, y1_ref, a0_ref, a1_ref, w1_ref, b1_ref, w2_ref,
                   b2_ref, batch_ref, z_ref):
    h = jnp.concatenate(
        [y0_ref[...] + a0_ref[...], y1_ref[...] + a1_ref[...]], axis=1)
    t = jnp.maximum(h @ w1_ref[...] + b1_ref[...], 0.0)
    o = jnp.maximum(t @ w2_ref[...] + b2_ref[...], 0.0)
    b = batch_ref[...].reshape(1, NB)
    oh = (b == lax.broadcasted_iota(jnp.int32, (G, NB), 0)).astype(jnp.float32)
    zp = oh @ o

    @pl.when(pl.program_id(0) == 0)
    def _():
        z_ref[...] = jnp.zeros_like(z_ref)

    z_ref[...] += zp


_mlp_pool = pl.pallas_call(
    _mlp_pool_body,
    grid=(GRID,),
    in_specs=[
        pl.BlockSpec((NB, DH), lambda i: (i, 0)),
        pl.BlockSpec((NB, DH), lambda i: (i, 0)),
        pl.BlockSpec((NB, DH), lambda i: (i, 0)),
        pl.BlockSpec((NB, DH), lambda i: (i, 0)),
        pl.BlockSpec((D, H), lambda i: (0, 0)),
        pl.BlockSpec((1, H), lambda i: (0, 0)),
        pl.BlockSpec((H, D), lambda i: (0, 0)),
        pl.BlockSpec((1, D), lambda i: (0, 0)),
        pl.BlockSpec((1, 1, NB), lambda i: (i, 0, 0)),
    ],
    out_specs=pl.BlockSpec((G, D), lambda i: (0, 0)),
    out_shape=jax.ShapeDtypeStruct((G, D), jnp.float32),
    compiler_params=pltpu.CompilerParams(
        dimension_semantics=("arbitrary",)),
)


def kernel(x, edge_index, batch, emb1, emb2,
           w1_l0, b1_l0, w2_l0, b2_l0,
           w1_l1, b1_l1, w2_l1, b2_l1,
           w1_l2, b1_l2, w2_l2, b2_l2):
    src = edge_index[0].astype(jnp.int32).reshape(TILES, CHUNKS, CHUNK)
    dst = edge_index[1].astype(jnp.int32).reshape(TILES, CHUNKS, CHUNK)
    zero = jnp.zeros((N, DH), jnp.float32)
    batch3 = batch.astype(jnp.int32).reshape(GRID, 1, NB)

    y0, y1 = _embed(x.astype(jnp.int32), emb1, emb2)

    layers = [(w1_l0, b1_l0, w2_l0, b2_l0),
              (w1_l1, b1_l1, w2_l1, b2_l1),
              (w1_l2, b1_l2, w2_l2, b2_l2)]
    sc_agg = _get_sc_agg()
    for li, (w1, b1, w2, b2) in enumerate(layers):
        a0, a1 = sc_agg(y0, y1, src, dst, zero)
        b1r = b1.reshape(1, H)
        b2r = b2.reshape(1, D)
        if li < 2:
            y0, y1 = _mlp(y0, y1, a0, a1, w1, b1r, w2, b2r)
        else:
            z = _mlp_pool(y0, y1, a0, a1, w1, b1r, w2, b2r, batch3)
    return z
